# baseline (speedup 1.0000x reference)
import functools

import jax
import jax.numpy as jnp
from jax import lax
from jax.experimental import pallas as pl
from jax.experimental.pallas import tpu as pltpu
from jax.experimental.pallas import tpu_sc as plsc

NC = 2
NS = 16
L = 16
NW = NC * NS


def _pad_up(v, m):
    return ((v + m - 1) // m) * m


def _sc_segsum(edges, u, NP, RS):
    E = edges.shape[1]
    EPT = E // NW
    WIN = _pad_up(EPT + 112, 128)
    gather = u is not None

    mesh = plsc.VectorSubcoreMesh(core_axis_name="c", subcore_axis_name="s",
                                  num_cores=NC)

    scratch = [
        pltpu.VMEM((2, WIN), jnp.int32),
        pltpu.VMEM((NP,), jnp.float32),
    ]
    if gather:
        scratch.append(pltpu.VMEM((NP,), jnp.float32))
    scratch.append(pltpu.SemaphoreType.DMA)

    @functools.partial(
        pl.kernel,
        out_type=jax.ShapeDtypeStruct((NW * RS,), jnp.float32),
        mesh=mesh,
        scratch_types=scratch,
        compiler_params=pltpu.CompilerParams(needs_layout_passes=False),
    )
    def seg_kernel(*refs):
        if gather:
            edges_hbm, u_hbm, out_hbm, e_v, acc_v, u_v, sem = refs
        else:
            edges_hbm, out_hbm, e_v, acc_v, sem = refs
        cid = lax.axis_index("c")
        sid = lax.axis_index("s")
        wid = sid * NC + cid

        begin = wid * EPT
        start = jnp.minimum((begin // 128) * 128, E - WIN)
        off = begin - start

        copies = [pltpu.async_copy(edges_hbm.at[:, pl.ds(start, WIN)],
                                   e_v, sem)]
        if gather:
            copies.append(pltpu.async_copy(u_hbm, u_v, sem))

        def zero_body(i, _):
            acc_v[pl.ds(i * L, L)] = jnp.zeros((L,), jnp.float32)
            return 0
        lax.fori_loop(0, NP // L, zero_body, 0, unroll=16)

        for c in copies:
            c.wait()

        ones = jnp.full((L,), 1.0, jnp.float32)

        def scat_body(i, _):
            d = e_v[1, pl.ds(off + i * L, L)]
            if gather:
                s = e_v[0, pl.ds(off + i * L, L)]
                vals = plsc.load_gather(u_v, [s])
            else:
                vals = ones
            plsc.addupdate_scatter(acc_v, [d], vals)
            return 0
        lax.fori_loop(0, EPT // L, scat_body, 0, unroll=32)

        pltpu.sync_copy(acc_v, out_hbm.at[pl.ds(wid * RS, NP)])

    if gather:
        return seg_kernel(edges, u)
    return seg_kernel(edges)


def _rowsum(flat_ref, NP, RS):
    acc = flat_ref[pl.ds(0, NP)]
    for r in range(1, NW):
        acc = acc + flat_ref[pl.ds(r * RS, NP)]
    return acc


def _vt(WdT_ref, W2_ref):
    return lax.dot_general(WdT_ref[...], W2_ref[...],
                           (((1,), (1,)), ((), ())),
                           preferred_element_type=jnp.float32)


def _tc0(x, W1, W2, WdT):
    NP = x.shape[0]

    def body(x_ref, W1_ref, W2_ref, WdT_ref, z_ref):
        vT = _vt(WdT_ref, W2_ref)
        wT = lax.dot_general(vT, W1_ref[...], (((1,), (1,)), ((), ())),
                             preferred_element_type=jnp.float32)
        z_ref[...] = jnp.sum(x_ref[...] * wT, axis=1)

    return pl.pallas_call(
        body,
        out_shape=jax.ShapeDtypeStruct((NP,), jnp.float32),
    )(x, W1, W2, WdT)


def _tc1(z, degp, RS):
    NP = z.shape[0]

    def body(z_ref, degp_ref, u_ref, norm_ref):
        deg = _rowsum(degp_ref, NP, RS) + 1.0
        norm = lax.rsqrt(deg)
        norm_ref[...] = norm
        u_ref[...] = norm * z_ref[...]

    return pl.pallas_call(
        body,
        out_shape=[jax.ShapeDtypeStruct((NP,), jnp.float32),
                   jax.ShapeDtypeStruct((NP,), jnp.float32)],
    )(z, degp)


def _tc2(s1p, u, norm, W2, WdT, b1r, RS):
    NP = u.shape[0]

    def body(s1p_ref, u_ref, norm_ref, W2_ref, WdT_ref, b1r_ref, u2_ref):
        vT = _vt(WdT_ref, W2_ref)
        c1 = jnp.sum(vT * b1r_ref[...])
        s1 = _rowsum(s1p_ref, NP, RS)
        y1 = norm_ref[...] * (s1 + u_ref[...]) + c1
        u2_ref[...] = norm_ref[...] * y1

    return pl.pallas_call(
        body,
        out_shape=jax.ShapeDtypeStruct((NP,), jnp.float32),
    )(s1p, u, norm, W2, WdT, b1r)


def _tc3(s2p, u2, norm, WdT, b2r, bdr, RS):
    NP = u2.shape[0]

    def body(s2p_ref, u2_ref, norm_ref, WdT_ref, b2r_ref, bdr_ref, out_ref):
        c2 = jnp.sum(WdT_ref[...] * b2r_ref[...]) + jnp.sum(bdr_ref[...])
        s2 = _rowsum(s2p_ref, NP, RS)
        out_ref[...] = norm_ref[...] * (s2 + u2_ref[...]) + c2

    return pl.pallas_call(
        body,
        out_shape=jax.ShapeDtypeStruct((NP,), jnp.float32),
    )(s2p, u2, norm, WdT, b2r, bdr)


def kernel(x, edge_index, W1, b1, W2, b2, Wd, bd):
    N = x.shape[0]
    E = edge_index.shape[1]

    edges = edge_index.astype(jnp.int32)

    Ep = _pad_up(E, max(NW * L, 128))
    NP = N if (N % L == 0 and Ep == E) else _pad_up(N + 1, L)
    if Ep > E:
        pad = jnp.zeros((2, Ep - E), jnp.int32).at[1].set(N)
        edges = jnp.concatenate([edges, pad], axis=1)
    if NP > N:
        x = jnp.zeros((NP, x.shape[1]), jnp.float32).at[:N].set(x)
    RS = _pad_up(NP, 128)

    WdT = Wd.T
    b1r = b1.reshape(1, -1)
    b2r = b2.reshape(1, -1)
    bdr = bd.reshape(1, -1)

    z = _tc0(x, W1, W2, WdT)
    degp = _sc_segsum(edges, None, NP, RS)
    u, norm = _tc1(z, degp, RS)
    s1p = _sc_segsum(edges, u, NP, RS)
    u2 = _tc2(s1p, u, norm, W2, WdT, b1r, RS)
    s2p = _sc_segsum(edges, u2, NP, RS)
    o = _tc3(s2p, u2, norm, WdT, b2r, bdr, RS)
    return o[:N, None]


# speedup vs baseline: 1.0125x; 1.0125x over previous
import functools

import jax
import jax.numpy as jnp
from jax import lax
from jax.experimental import pallas as pl
from jax.experimental.pallas import tpu as pltpu
from jax.experimental.pallas import tpu_sc as plsc

NC = 2
NS = 16
L = 16
NW = NC * NS


def _pad_up(v, m):
    return ((v + m - 1) // m) * m


def _sc_segsum(edges, u, NP, RS):
    E = edges.shape[1]
    EPT = E // NW
    WIN = _pad_up(EPT + 112, 128)
    gather = u is not None

    mesh = plsc.VectorSubcoreMesh(core_axis_name="c", subcore_axis_name="s",
                                  num_cores=NC)

    scratch = [
        pltpu.VMEM((2, WIN), jnp.int32),
        pltpu.VMEM((NP,), jnp.float32),
    ]
    if gather:
        scratch.append(pltpu.VMEM((NP,), jnp.float32))
    scratch.append(pltpu.SemaphoreType.DMA)

    @functools.partial(
        pl.kernel,
        out_type=jax.ShapeDtypeStruct((NW * RS,), jnp.float32),
        mesh=mesh,
        scratch_types=scratch,
        compiler_params=pltpu.CompilerParams(needs_layout_passes=False),
    )
    def seg_kernel(*refs):
        if gather:
            edges_hbm, u_hbm, out_hbm, e_v, acc_v, u_v, sem = refs
        else:
            edges_hbm, out_hbm, e_v, acc_v, sem = refs
        cid = lax.axis_index("c")
        sid = lax.axis_index("s")
        wid = sid * NC + cid

        begin = wid * EPT
        start = jnp.minimum((begin // 128) * 128, E - WIN)
        off = begin - start

        copies = [pltpu.async_copy(edges_hbm.at[:, pl.ds(start, WIN)],
                                   e_v, sem)]
        if gather:
            copies.append(pltpu.async_copy(u_hbm, u_v, sem))

        def zero_body(i, _):
            acc_v[pl.ds(i * L, L)] = jnp.zeros((L,), jnp.float32)
            return 0
        lax.fori_loop(0, NP // L, zero_body, 0, unroll=8)

        for c in copies:
            c.wait()

        ones = jnp.full((L,), 1.0, jnp.float32)

        def scat_body(i, _):
            d = e_v[1, pl.ds(off + i * L, L)]
            if gather:
                s = e_v[0, pl.ds(off + i * L, L)]
                vals = plsc.load_gather(u_v, [s])
            else:
                vals = ones
            plsc.addupdate_scatter(acc_v, [d], vals)
            return 0
        lax.fori_loop(0, EPT // L, scat_body, 0, unroll=16)

        pltpu.sync_copy(acc_v, out_hbm.at[pl.ds(wid * RS, NP)])

    if gather:
        return seg_kernel(edges, u)
    return seg_kernel(edges)


def _rowsum(flat_ref, NP, RS):
    acc = flat_ref[pl.ds(0, NP)]
    for r in range(1, NW):
        acc = acc + flat_ref[pl.ds(r * RS, NP)]
    return acc


def _vt(WdT_ref, W2_ref):
    return lax.dot_general(WdT_ref[...], W2_ref[...],
                           (((1,), (1,)), ((), ())),
                           preferred_element_type=jnp.float32)


def _tc0(x, W1, W2, WdT):
    NP = x.shape[0]

    def body(x_ref, W1_ref, W2_ref, WdT_ref, z_ref):
        vT = _vt(WdT_ref, W2_ref)
        wT = lax.dot_general(vT, W1_ref[...], (((1,), (1,)), ((), ())),
                             preferred_element_type=jnp.float32)
        z_ref[...] = jnp.sum(x_ref[...] * wT, axis=1)

    return pl.pallas_call(
        body,
        out_shape=jax.ShapeDtypeStruct((NP,), jnp.float32),
    )(x, W1, W2, WdT)


def _tc1(z, degp, RS):
    NP = z.shape[0]

    def body(z_ref, degp_ref, u_ref, norm_ref):
        deg = _rowsum(degp_ref, NP, RS) + 1.0
        norm = lax.rsqrt(deg)
        norm_ref[...] = norm
        u_ref[...] = norm * z_ref[...]

    return pl.pallas_call(
        body,
        out_shape=[jax.ShapeDtypeStruct((NP,), jnp.float32),
                   jax.ShapeDtypeStruct((NP,), jnp.float32)],
    )(z, degp)


def _tc2(s1p, u, norm, W2, WdT, b1r, RS):
    NP = u.shape[0]

    def body(s1p_ref, u_ref, norm_ref, W2_ref, WdT_ref, b1r_ref, u2_ref):
        vT = _vt(WdT_ref, W2_ref)
        c1 = jnp.sum(vT * b1r_ref[...])
        s1 = _rowsum(s1p_ref, NP, RS)
        y1 = norm_ref[...] * (s1 + u_ref[...]) + c1
        u2_ref[...] = norm_ref[...] * y1

    return pl.pallas_call(
        body,
        out_shape=jax.ShapeDtypeStruct((NP,), jnp.float32),
    )(s1p, u, norm, W2, WdT, b1r)


def _tc3(s2p, u2, norm, WdT, b2r, bdr, RS):
    NP = u2.shape[0]

    def body(s2p_ref, u2_ref, norm_ref, WdT_ref, b2r_ref, bdr_ref, out_ref):
        c2 = jnp.sum(WdT_ref[...] * b2r_ref[...]) + jnp.sum(bdr_ref[...])
        s2 = _rowsum(s2p_ref, NP, RS)
        out_ref[...] = norm_ref[...] * (s2 + u2_ref[...]) + c2

    return pl.pallas_call(
        body,
        out_shape=jax.ShapeDtypeStruct((NP,), jnp.float32),
    )(s2p, u2, norm, WdT, b2r, bdr)


def kernel(x, edge_index, W1, b1, W2, b2, Wd, bd):
    N = x.shape[0]
    E = edge_index.shape[1]

    edges = edge_index.astype(jnp.int32)

    Ep = _pad_up(E, max(NW * L, 128))
    NP = N if (N % L == 0 and Ep == E) else _pad_up(N + 1, L)
    if Ep > E:
        pad = jnp.zeros((2, Ep - E), jnp.int32).at[1].set(N)
        edges = jnp.concatenate([edges, pad], axis=1)
    if NP > N:
        x = jnp.zeros((NP, x.shape[1]), jnp.float32).at[:N].set(x)
    RS = _pad_up(NP, 128)

    WdT = Wd.T
    b1r = b1.reshape(1, -1)
    b2r = b2.reshape(1, -1)
    bdr = bd.reshape(1, -1)

    z = _tc0(x, W1, W2, WdT)
    degp = _sc_segsum(edges, None, NP, RS)
    u, norm = _tc1(z, degp, RS)
    s1p = _sc_segsum(edges, u, NP, RS)
    u2 = _tc2(s1p, u, norm, W2, WdT, b1r, RS)
    s2p = _sc_segsum(edges, u2, NP, RS)
    o = _tc3(s2p, u2, norm, WdT, b2r, bdr, RS)
    return o[:N, None]
